# baseline (speedup 1.0000x reference)
import functools

import jax
import jax.numpy as jnp
from jax.experimental import pallas as pl
from jax.experimental.pallas import tpu as pltpu
from jax.experimental.pallas import tpu_sc as plsc

N = 10000
E = 320000
NC = 2
NS = 16
NW = NC * NS
K = 80
NPS = 632
N_ACC = NS * NPS
ROW_BLK = 400


R = 4
LAG = 2


def _make_sc_agg(H, E_pad):
    per_tile = E_pad // NW
    n_chunks = per_tile // K
    assert n_chunks % R == 0 and n_chunks >= 2 * R
    mesh = plsc.VectorSubcoreMesh(core_axis_name="c", subcore_axis_name="s")

    @functools.partial(
        pl.kernel,
        mesh=mesh,
        out_type=jax.ShapeDtypeStruct((NC, N_ACC, H), jnp.float32),
        scratch_types=[
            pltpu.VMEM((R, K), jnp.int32),
            pltpu.VMEM((R, K), jnp.int32),
            pltpu.VMEM((R, K, H), jnp.float32),
            pltpu.VMEM_SHARED((N_ACC, H), jnp.float32),
        ] + [pltpu.SemaphoreType.DMA] * (4 * R),
    )
    def agg_kernel(hs, src, dst, zeros, out, src_v, dst_v, rows_v, acc,
                   *sems):
        c = jax.lax.axis_index("c")
        s = jax.lax.axis_index("s")
        wid = c * NS + s
        sem_ls = sems[0:R]
        sem_ld = sems[R:2 * R]
        sem_g = sems[2 * R:3 * R]
        sem_s = sems[3 * R:4 * R]
        pltpu.sync_copy(zeros.at[pl.ds(s * NPS, NPS)],
                        acc.at[pl.ds(s * NPS, NPS)])
        plsc.subcore_barrier()
        base = wid * per_tile

        def cp_ls(j, x):
            return pltpu.make_async_copy(
                src.at[pl.ds(base + j * K, K)], src_v.at[x], sem_ls[x])

        def cp_ld(j, x):
            return pltpu.make_async_copy(
                dst.at[pl.ds(base + j * K, K)], dst_v.at[x], sem_ld[x])

        def cp_g(x):
            return pltpu.make_async_copy(
                hs.at[src_v.at[x]], rows_v.at[x], sem_g[x])

        def scat_start(x):
            pltpu.async_copy(rows_v.at[x], acc.at[dst_v.at[x]], sem_s[x],
                             add=True)

        def scat_wait(x):
            pltpu.make_async_copy(rows_v.at[x], acc.at[dst_v.at[x]],
                                  sem_s[x]).wait()

        cp_ls(0, 0).start()

        def quad(t, carry):
            for u in range(R):
                j = R * t + u
                x = u
                a = (u - LAG) % R

                def lagged():
                    cp_g(a).wait()
                    cp_ld(j - LAG, a).wait()
                    scat_start(a)

                if u >= LAG:
                    lagged()
                else:
                    @pl.when(t > 0)
                    def _():
                        lagged()

                @pl.when(t > 0)
                def _():
                    scat_wait(x)

                cp_ld(j, x).start()
                if u == R - 1:
                    @pl.when(t < n_chunks // R - 1)
                    def _():
                        cp_ls(j + 1, (u + 1) % R).start()
                else:
                    cp_ls(j + 1, (u + 1) % R).start()
                cp_ls(j, x).wait()
                cp_g(x).start()
            return carry

        jax.lax.fori_loop(0, n_chunks // R, quad, 0)
        n = n_chunks
        for j in (n - LAG, n - 1):
            x = j % R
            cp_g(x).wait()
            cp_ld(j, x).wait()
            scat_start(x)
        for j in range(n - R, n):
            scat_wait(j % R)
        plsc.subcore_barrier()
        pltpu.sync_copy(acc.at[pl.ds(s * NPS, NPS)],
                        out.at[c, pl.ds(s * NPS, NPS)])

    return agg_kernel


_EQ = NW * K * R
_VIEW_E_PAD = ((E + _EQ - 1) // _EQ) * _EQ
_TRUNK_E_PAD = ((E + N + _EQ - 1) // _EQ) * _EQ
_AGG_VIEW = _make_sc_agg(128, _VIEW_E_PAD)
_AGG_TRUNK = _make_sc_agg(128, _TRUNK_E_PAD)

DEG_SEGS = 8
KD = 128
B_CH = 16
_SEG_LENS = [E] * 6 + [E + N] * 2
_SEG_QUANT = NW * KD * B_CH
_SEG_PADDED = [((l + _SEG_QUANT - 1) // _SEG_QUANT) * _SEG_QUANT
               for l in _SEG_LENS]
_SEG_BLOCKS = [p // _SEG_QUANT for p in _SEG_PADDED]
_DEG_CHUNKS = sum(_SEG_PADDED) // (NW * KD)


def _make_sc_deg():
    mesh = plsc.VectorSubcoreMesh(core_axis_name="c", subcore_axis_name="s")

    @functools.partial(
        pl.kernel,
        mesh=mesh,
        out_type=jax.ShapeDtypeStruct((NC, N_ACC, 128), jnp.float32),
        scratch_types=[
            pltpu.VMEM((KD, 128), jnp.float32),
            pltpu.VMEM((2, B_CH, KD), jnp.int32),
            pltpu.VMEM_SHARED((N_ACC, 128), jnp.float32),
            pltpu.SemaphoreType.DMA,
            pltpu.SemaphoreType.DMA,
            pltpu.SemaphoreType.DMA,
            pltpu.SemaphoreType.DMA,
        ],
    )
    def deg_kernel(idx3, pat, zeros, out, pat_v, idx_v, acc,
                   sem_i0, sem_i1, sem_s0, sem_s1):
        c = jax.lax.axis_index("c")
        s = jax.lax.axis_index("s")
        wid = c * NS + s
        sem_i = (sem_i0, sem_i1)
        sem_s = (sem_s0, sem_s1)
        pltpu.sync_copy(zeros.at[pl.ds(s * NPS, NPS)],
                        acc.at[pl.ds(s * NPS, NPS)])
        plsc.subcore_barrier()

        def idx_block(blk, buf, sem):
            return pltpu.make_async_copy(
                idx3.at[wid, pl.ds(blk * B_CH, B_CH)], idx_v.at[buf], sem)

        def fire(buf, sem):
            for j in range(B_CH):
                pltpu.async_copy(pat_v, acc.at[idx_v.at[buf, j]], sem,
                                 add=True)

        def drain(buf, sem):
            for j in range(B_CH):
                pltpu.make_async_copy(pat_v, acc.at[idx_v.at[buf, j]],
                                      sem).wait()

        blk_base = 0
        for g, nblk in enumerate(_SEG_BLOCKS):
            pltpu.sync_copy(pat.at[g], pat_v)
            idx_block(blk_base, 0, sem_i0).start()

            def seg_body(t, carry, blk_base=blk_base, nblk=nblk):
                for u in (0, 1):
                    blk = 2 * t + u
                    nxt = blk + 1

                    @pl.when(blk > 0)
                    def _():
                        drain(1 - u, sem_s[1 - u])

                    @pl.when(nxt < nblk)
                    def _():
                        idx_block(blk_base + nxt, 1 - u, sem_i[1 - u]).start()

                    idx_block(blk_base + blk, u, sem_i[u]).wait()
                    fire(u, sem_s[u])
                return carry

            if nblk % 2 == 0:
                jax.lax.fori_loop(0, nblk // 2, seg_body, 0)
                last = 1
            else:
                jax.lax.fori_loop(0, nblk // 2, seg_body, 0)
                t = nblk // 2
                blk = 2 * t

                @pl.when(blk > 0)
                def _():
                    drain(1, sem_s1)

                idx_block(blk_base + blk, 0, sem_i0).wait()
                fire(0, sem_s0)
                last = 0
            drain(last, sem_s[last])
            blk_base += nblk

        plsc.subcore_barrier()
        pltpu.sync_copy(acc.at[pl.ds(s * NPS, NPS)],
                        out.at[c, pl.ds(s * NPS, NPS)])

    return deg_kernel


_DEG_KERNEL = _make_sc_deg()


def _deg_patterns():
    pat = jnp.zeros((DEG_SEGS, KD, 128), jnp.float32)
    for g in range(DEG_SEGS):
        pat = pat.at[g, :, 16 * g:16 * (g + 1)].set(1.0)
    return pat


def _all_degrees(segs):
    cols = []
    for g, (seg, padded) in enumerate(zip(segs, _SEG_PADDED)):
        pad = padded - seg.shape[0]
        col = jnp.concatenate([seg.astype(jnp.int32),
                               jnp.full((pad,), N, jnp.int32)])
        cols.append(col.reshape(NW, padded // (NW * KD), KD))
    idx3 = jnp.concatenate(cols, axis=1)
    zeros = jnp.zeros((N_ACC, 128), jnp.float32)
    parts = _DEG_KERNEL(idx3, _deg_patterns(), zeros)
    deg = parts[0, :N] + parts[1, :N]
    return deg.reshape(N, DEG_SEGS, 16)[:, :, 0].T


def _norms(deg):
    return jnp.where(deg > 0, jax.lax.rsqrt(jnp.maximum(deg, 1e-12)), 0.0)


def _sc_aggregate(agg_fn, e_pad, hs, src, dst):
    h = hs.shape[1]
    if h < 128:
        hs = jnp.pad(hs, ((0, 0), (0, 128 - h)))
    pad = e_pad - src.shape[0]
    if pad:
        src = jnp.concatenate([src, jnp.zeros((pad,), src.dtype)])
        dst = jnp.concatenate([dst, jnp.full((pad,), N, dst.dtype)])
    zeros = jnp.zeros((N_ACC, 128), hs.dtype)
    parts = agg_fn(hs, src, dst, zeros)
    return parts[0, :N, :h] + parts[1, :N, :h]


def _decoder_body(xh_blk_ref, xh_all_ref, out_ref):
    a = xh_blk_ref[...]
    b = xh_all_ref[...]
    out_ref[...] = jax.lax.dot_general(
        a, b, (((1,), (1,)), ((), ())), preferred_element_type=jnp.float32)


def _decoder(xh):
    n = xh.shape[0]
    return pl.pallas_call(
        _decoder_body,
        grid=(n // ROW_BLK,),
        in_specs=[
            pl.BlockSpec((ROW_BLK, xh.shape[1]), lambda i: (i, 0)),
            pl.BlockSpec((n, xh.shape[1]), lambda i: (0, 0)),
        ],
        out_specs=pl.BlockSpec((ROW_BLK, n), lambda i: (i, 0)),
        out_shape=jax.ShapeDtypeStruct((n, n), jnp.float32),
    )(xh, xh)


def _gcn_pair(x, src, dst, W0, b0, W1, b1, agg0, agg1, ns, nd):
    hs = (x @ W0) * ns[:, None]
    agg = _sc_aggregate(agg0, _VIEW_E_PAD, hs, src, dst)
    h1 = jax.nn.relu(agg * nd[:, None] + b0)
    hs = (h1 @ W1) * ns[:, None]
    agg = _sc_aggregate(agg1, _VIEW_E_PAD, hs, src, dst)
    return agg * nd[:, None] + b1


def kernel(feature0, feature1, feature2, W00, b00, W01, b01, W10, b10, W11,
           b11, W20, b20, W21, b21, Wm0, bm0, Wm1, bm1, Wf0, Wf1, Wf2, Wd,
           edge_index0, edge_index1, edge_index2, edge_index):
    n = feature0.shape[0]
    loops = jnp.arange(n, dtype=edge_index.dtype)
    src_t = jnp.concatenate([edge_index[0], loops])
    dst_t = jnp.concatenate([edge_index[1], loops])
    deg = _all_degrees([edge_index0[0], edge_index0[1],
                        edge_index1[0], edge_index1[1],
                        edge_index2[0], edge_index2[1],
                        src_t, dst_t])
    norms = _norms(deg)

    h0 = _gcn_pair(feature0, edge_index0[0], edge_index0[1], W00, b00, W01,
                   b01, _AGG_VIEW, _AGG_VIEW, norms[0], norms[1])
    h1 = _gcn_pair(feature1, edge_index1[0], edge_index1[1], W10, b10, W11,
                   b11, _AGG_VIEW, _AGG_VIEW, norms[2], norms[3])
    h2 = _gcn_pair(feature2, edge_index2[0], edge_index2[1], W20, b20, W21,
                   b21, _AGG_VIEW, _AGG_VIEW, norms[4], norms[5])

    t = h0 @ Wf0 + h1 @ Wf1 + h2 @ Wf2
    xh = jax.nn.relu(jax.nn.softmax(t, axis=1))

    ns, nd = norms[6], norms[7]
    hs = (xh @ Wm0) * ns[:, None]
    agg = _sc_aggregate(_AGG_TRUNK, _TRUNK_E_PAD, hs, src_t, dst_t)
    xh = jax.nn.relu(agg * nd[:, None] + bm0)
    hs = (xh @ Wm1) * ns[:, None]
    agg = _sc_aggregate(_AGG_TRUNK, _TRUNK_E_PAD, hs, src_t, dst_t)
    xh = agg * nd[:, None] + bm1

    adj = _decoder(xh)
    return (adj, adj, adj, xh)


# speedup vs baseline: 1.2940x; 1.2940x over previous
import functools

import jax
import jax.numpy as jnp
from jax.experimental import pallas as pl
from jax.experimental.pallas import tpu as pltpu
from jax.experimental.pallas import tpu_sc as plsc

N = 10000
E = 320000
NC = 2
NS = 16
NW = NC * NS
K = 80
NPS = 632
N_ACC = NS * NPS
ROW_BLK = 400


R = 2
LAG = 1


def _make_sc_agg(H, E_pad):
    per_tile = E_pad // NW
    n_chunks = per_tile // K
    assert n_chunks % R == 0 and n_chunks >= 2 * R
    mesh = plsc.VectorSubcoreMesh(core_axis_name="c", subcore_axis_name="s")

    @functools.partial(
        pl.kernel,
        mesh=mesh,
        out_type=jax.ShapeDtypeStruct((NC, N_ACC, H), jnp.float32),
        scratch_types=[
            pltpu.VMEM((R, K), jnp.int32),
            pltpu.VMEM((R, K), jnp.int32),
            pltpu.VMEM((R, K, H), jnp.float32),
            pltpu.VMEM_SHARED((N_ACC, H), jnp.float32),
        ] + [pltpu.SemaphoreType.DMA] * (4 * R),
    )
    def agg_kernel(hs, src, dst, zeros, out, src_v, dst_v, rows_v, acc,
                   *sems):
        c = jax.lax.axis_index("c")
        s = jax.lax.axis_index("s")
        wid = c * NS + s
        sem_ls = sems[0:R]
        sem_ld = sems[R:2 * R]
        sem_g = sems[2 * R:3 * R]
        sem_s = sems[3 * R:4 * R]
        pltpu.sync_copy(zeros.at[pl.ds(s * NPS, NPS)],
                        acc.at[pl.ds(s * NPS, NPS)])
        plsc.subcore_barrier()
        base = wid * per_tile

        def cp_ls(j, x):
            return pltpu.make_async_copy(
                src.at[pl.ds(base + j * K, K)], src_v.at[x], sem_ls[x])

        def cp_ld(j, x):
            return pltpu.make_async_copy(
                dst.at[pl.ds(base + j * K, K)], dst_v.at[x], sem_ld[x])

        def cp_g(x):
            return pltpu.make_async_copy(
                hs.at[src_v.at[x]], rows_v.at[x], sem_g[x])

        def scat_start(x):
            pltpu.async_copy(rows_v.at[x], acc.at[dst_v.at[x]], sem_s[x],
                             add=True)

        def scat_wait(x):
            pltpu.make_async_copy(rows_v.at[x], acc.at[dst_v.at[x]],
                                  sem_s[x]).wait()

        cp_ls(0, 0).start()

        def quad(t, carry):
            for u in range(R):
                j = R * t + u
                x = u
                a = (u - LAG) % R

                def lagged():
                    cp_g(a).wait()
                    cp_ld(j - LAG, a).wait()
                    scat_start(a)

                if u >= LAG:
                    lagged()
                else:
                    @pl.when(t > 0)
                    def _():
                        lagged()

                @pl.when(t > 0)
                def _():
                    scat_wait(x)

                cp_ld(j, x).start()
                if u == R - 1:
                    @pl.when(t < n_chunks // R - 1)
                    def _():
                        cp_ls(j + 1, (u + 1) % R).start()
                else:
                    cp_ls(j + 1, (u + 1) % R).start()
                cp_ls(j, x).wait()
                cp_g(x).start()
            return carry

        jax.lax.fori_loop(0, n_chunks // R, quad, 0)
        n = n_chunks
        for j in range(n - LAG, n):
            x = j % R
            cp_g(x).wait()
            cp_ld(j, x).wait()
            scat_start(x)
        for j in range(n - R, n):
            scat_wait(j % R)
        plsc.subcore_barrier()
        pltpu.sync_copy(acc.at[pl.ds(s * NPS, NPS)],
                        out.at[c, pl.ds(s * NPS, NPS)])

    return agg_kernel


_EQ = NW * K * R
_VIEW_E_PAD = ((E + _EQ - 1) // _EQ) * _EQ
_TRUNK_E_PAD = ((E + N + _EQ - 1) // _EQ) * _EQ
_AGG_VIEW = _make_sc_agg(128, _VIEW_E_PAD)
_AGG_TRUNK = _make_sc_agg(128, _TRUNK_E_PAD)

DEG_SEGS = 8
KD = 128
B_CH = 16
_SEG_LENS = [E] * 6 + [E + N] * 2
_SEG_QUANT = NW * KD * B_CH
_SEG_PADDED = [((l + _SEG_QUANT - 1) // _SEG_QUANT) * _SEG_QUANT
               for l in _SEG_LENS]
_SEG_BLOCKS = [p // _SEG_QUANT for p in _SEG_PADDED]
_DEG_CHUNKS = sum(_SEG_PADDED) // (NW * KD)


def _make_sc_deg():
    mesh = plsc.VectorSubcoreMesh(core_axis_name="c", subcore_axis_name="s")

    @functools.partial(
        pl.kernel,
        mesh=mesh,
        out_type=jax.ShapeDtypeStruct((NC, N_ACC, 128), jnp.float32),
        scratch_types=[
            pltpu.VMEM((KD, 128), jnp.float32),
            pltpu.VMEM((2, B_CH, KD), jnp.int32),
            pltpu.VMEM_SHARED((N_ACC, 128), jnp.float32),
            pltpu.SemaphoreType.DMA,
            pltpu.SemaphoreType.DMA,
            pltpu.SemaphoreType.DMA,
            pltpu.SemaphoreType.DMA,
        ],
    )
    def deg_kernel(idx3, pat, zeros, out, pat_v, idx_v, acc,
                   sem_i0, sem_i1, sem_s0, sem_s1):
        c = jax.lax.axis_index("c")
        s = jax.lax.axis_index("s")
        wid = c * NS + s
        sem_i = (sem_i0, sem_i1)
        sem_s = (sem_s0, sem_s1)
        pltpu.sync_copy(zeros.at[pl.ds(s * NPS, NPS)],
                        acc.at[pl.ds(s * NPS, NPS)])
        plsc.subcore_barrier()

        def idx_block(blk, buf, sem):
            return pltpu.make_async_copy(
                idx3.at[wid, pl.ds(blk * B_CH, B_CH)], idx_v.at[buf], sem)

        def fire(buf, sem):
            for j in range(B_CH):
                pltpu.async_copy(pat_v, acc.at[idx_v.at[buf, j]], sem,
                                 add=True)

        def drain(buf, sem):
            for j in range(B_CH):
                pltpu.make_async_copy(pat_v, acc.at[idx_v.at[buf, j]],
                                      sem).wait()

        blk_base = 0
        for g, nblk in enumerate(_SEG_BLOCKS):
            pltpu.sync_copy(pat.at[g], pat_v)
            idx_block(blk_base, 0, sem_i0).start()

            def seg_body(t, carry, blk_base=blk_base, nblk=nblk):
                for u in (0, 1):
                    blk = 2 * t + u
                    nxt = blk + 1

                    @pl.when(blk > 0)
                    def _():
                        drain(1 - u, sem_s[1 - u])

                    @pl.when(nxt < nblk)
                    def _():
                        idx_block(blk_base + nxt, 1 - u, sem_i[1 - u]).start()

                    idx_block(blk_base + blk, u, sem_i[u]).wait()
                    fire(u, sem_s[u])
                return carry

            if nblk % 2 == 0:
                jax.lax.fori_loop(0, nblk // 2, seg_body, 0)
                last = 1
            else:
                jax.lax.fori_loop(0, nblk // 2, seg_body, 0)
                t = nblk // 2
                blk = 2 * t

                @pl.when(blk > 0)
                def _():
                    drain(1, sem_s1)

                idx_block(blk_base + blk, 0, sem_i0).wait()
                fire(0, sem_s0)
                last = 0
            drain(last, sem_s[last])
            blk_base += nblk

        plsc.subcore_barrier()
        pltpu.sync_copy(acc.at[pl.ds(s * NPS, NPS)],
                        out.at[c, pl.ds(s * NPS, NPS)])

    return deg_kernel


_DEG_KERNEL = _make_sc_deg()


def _deg_patterns():
    pat = jnp.zeros((DEG_SEGS, KD, 128), jnp.float32)
    for g in range(DEG_SEGS):
        pat = pat.at[g, :, 16 * g:16 * (g + 1)].set(1.0)
    return pat


def _all_degrees(segs):
    cols = []
    for g, (seg, padded) in enumerate(zip(segs, _SEG_PADDED)):
        pad = padded - seg.shape[0]
        col = jnp.concatenate([seg.astype(jnp.int32),
                               jnp.full((pad,), N, jnp.int32)])
        cols.append(col.reshape(NW, padded // (NW * KD), KD))
    idx3 = jnp.concatenate(cols, axis=1)
    zeros = jnp.zeros((N_ACC, 128), jnp.float32)
    parts = _DEG_KERNEL(idx3, _deg_patterns(), zeros)
    deg = parts[0, :N] + parts[1, :N]
    return deg.reshape(N, DEG_SEGS, 16)[:, :, 0].T


def _norms(deg):
    return jnp.where(deg > 0, jax.lax.rsqrt(jnp.maximum(deg, 1e-12)), 0.0)


def _sc_aggregate(agg_fn, e_pad, hs, src, dst):
    h = hs.shape[1]
    if h < 128:
        hs = jnp.pad(hs, ((0, 0), (0, 128 - h)))
    pad = e_pad - src.shape[0]
    if pad:
        src = jnp.concatenate([src, jnp.zeros((pad,), src.dtype)])
        dst = jnp.concatenate([dst, jnp.full((pad,), N, dst.dtype)])
    zeros = jnp.zeros((N_ACC, 128), hs.dtype)
    parts = agg_fn(hs, src, dst, zeros)
    return parts[0, :N, :h] + parts[1, :N, :h]


def _decoder_body(xh_blk_ref, xh_all_ref, out_ref):
    a = xh_blk_ref[...]
    b = xh_all_ref[...]
    out_ref[...] = jax.lax.dot_general(
        a, b, (((1,), (1,)), ((), ())), preferred_element_type=jnp.float32)


def _decoder(xh):
    n = xh.shape[0]
    return pl.pallas_call(
        _decoder_body,
        grid=(n // ROW_BLK,),
        in_specs=[
            pl.BlockSpec((ROW_BLK, xh.shape[1]), lambda i: (i, 0)),
            pl.BlockSpec((n, xh.shape[1]), lambda i: (0, 0)),
        ],
        out_specs=pl.BlockSpec((ROW_BLK, n), lambda i: (i, 0)),
        out_shape=jax.ShapeDtypeStruct((n, n), jnp.float32),
    )(xh, xh)


def _gcn_pair(x, src, dst, W0, b0, W1, b1, agg0, agg1, ns, nd):
    hs = (x @ W0) * ns[:, None]
    agg = _sc_aggregate(agg0, _VIEW_E_PAD, hs, src, dst)
    h1 = jax.nn.relu(agg * nd[:, None] + b0)
    hs = (h1 @ W1) * ns[:, None]
    agg = _sc_aggregate(agg1, _VIEW_E_PAD, hs, src, dst)
    return agg * nd[:, None] + b1


def kernel(feature0, feature1, feature2, W00, b00, W01, b01, W10, b10, W11,
           b11, W20, b20, W21, b21, Wm0, bm0, Wm1, bm1, Wf0, Wf1, Wf2, Wd,
           edge_index0, edge_index1, edge_index2, edge_index):
    n = feature0.shape[0]
    loops = jnp.arange(n, dtype=edge_index.dtype)
    src_t = jnp.concatenate([edge_index[0], loops])
    dst_t = jnp.concatenate([edge_index[1], loops])
    deg = _all_degrees([edge_index0[0], edge_index0[1],
                        edge_index1[0], edge_index1[1],
                        edge_index2[0], edge_index2[1],
                        src_t, dst_t])
    norms = _norms(deg)

    h0 = _gcn_pair(feature0, edge_index0[0], edge_index0[1], W00, b00, W01,
                   b01, _AGG_VIEW, _AGG_VIEW, norms[0], norms[1])
    h1 = _gcn_pair(feature1, edge_index1[0], edge_index1[1], W10, b10, W11,
                   b11, _AGG_VIEW, _AGG_VIEW, norms[2], norms[3])
    h2 = _gcn_pair(feature2, edge_index2[0], edge_index2[1], W20, b20, W21,
                   b21, _AGG_VIEW, _AGG_VIEW, norms[4], norms[5])

    t = h0 @ Wf0 + h1 @ Wf1 + h2 @ Wf2
    xh = jax.nn.relu(jax.nn.softmax(t, axis=1))

    ns, nd = norms[6], norms[7]
    hs = (xh @ Wm0) * ns[:, None]
    agg = _sc_aggregate(_AGG_TRUNK, _TRUNK_E_PAD, hs, src_t, dst_t)
    xh = jax.nn.relu(agg * nd[:, None] + bm0)
    hs = (xh @ Wm1) * ns[:, None]
    agg = _sc_aggregate(_AGG_TRUNK, _TRUNK_E_PAD, hs, src_t, dst_t)
    xh = agg * nd[:, None] + bm1

    adj = _decoder(xh)
    return (adj, adj, adj, xh)
